# baseline (speedup 1.0000x reference)
import functools

import jax
import jax.numpy as jnp
from jax import lax
from jax.experimental import pallas as pl
from jax.experimental.pallas import tpu as pltpu
from jax.experimental.pallas import tpu_sc as plsc

_B, _C, _T, _H, _W = 4, 96, 8, 14, 14
_K = 4
_N = _T * _H * _W
_C2 = 2 * _C
_C2P = 256
_BN = _B * _N
_KBN = _K * _BN
_RB = 224
_NRB = _N // _RB
_GW = 128


def _prep_body(xin_ref, pe_ref, av_ref, x_ref, xn_ref, sq_ref, v_ref):
    x = xin_ref[0] + pe_ref[0]
    x_ref[0] = x
    nrm = jnp.sqrt(jnp.sum(x * x, axis=1, keepdims=True))
    xn = x / jnp.maximum(nrm, 1e-12)
    xn_ref[0] = xn
    sq_ref[0] = jnp.sum(xn * xn, axis=1, keepdims=True)
    v_ref[0] = jnp.dot(x, av_ref[...], preferred_element_type=jnp.float32)


def _prep(xin, pe, av):
    return pl.pallas_call(
        _prep_body,
        grid=(_B,),
        in_specs=[
            pl.BlockSpec((1, _N, _C), lambda b: (b, 0, 0)),
            pl.BlockSpec((1, _N, _C), lambda b: (0, 0, 0)),
            pl.BlockSpec((_C, _C2P), lambda b: (0, 0)),
        ],
        out_specs=[
            pl.BlockSpec((1, _N, _C), lambda b: (b, 0, 0)),
            pl.BlockSpec((1, _N, _C), lambda b: (b, 0, 0)),
            pl.BlockSpec((1, _N, 1), lambda b: (b, 0, 0)),
            pl.BlockSpec((1, _N, _C2P), lambda b: (b, 0, 0)),
        ],
        out_shape=[
            jax.ShapeDtypeStruct((_B, _N, _C), jnp.float32),
            jax.ShapeDtypeStruct((_B, _N, _C), jnp.float32),
            jax.ShapeDtypeStruct((_B, _N, 1), jnp.float32),
            jax.ShapeDtypeStruct((_B, _N, _C2P), jnp.float32),
        ],
    )(xin, pe, av)


def _knn_body(xb_ref, xn_ref, sqt_ref, idx_ref):
    b = pl.program_id(0)
    xb = xb_ref[0]
    xn = xn_ref[0]
    dot = lax.dot_general(xb, xn, (((1,), (1,)), ((), ())),
                          preferred_element_type=jnp.float32)
    d = sqt_ref[0] - 2.0 * dot
    iot = lax.broadcasted_iota(jnp.int32, (_RB, _N), 1)
    cols = []
    for _ in range(_K):
        mv = jnp.min(d, axis=1, keepdims=True)
        ik = jnp.min(jnp.where(d == mv, iot, _N), axis=1, keepdims=True)
        cols.append(ik)
        d = jnp.where(iot == ik, jnp.inf, d)
    idx_ref[0] = jnp.concatenate(cols, axis=1) + b * _N


def _knn(xn, sqt):
    return pl.pallas_call(
        _knn_body,
        grid=(_B, _NRB),
        in_specs=[
            pl.BlockSpec((1, _RB, _C), lambda b, i: (b, i, 0)),
            pl.BlockSpec((1, _N, _C), lambda b, i: (b, 0, 0)),
            pl.BlockSpec((1, 1, _N), lambda b, i: (b, 0, 0)),
        ],
        out_specs=pl.BlockSpec((1, _RB, _K), lambda b, i: (b, i, 0)),
        out_shape=jax.ShapeDtypeStruct((_B, _N, _K), jnp.int32),
    )(xn, xn, sqt)


def _sc_gather(v_flat, idx_flat):
    idx2 = idx_flat.reshape(1, _KBN)
    mesh = plsc.VectorSubcoreMesh(core_axis_name="core", subcore_axis_name="subcore")

    @functools.partial(
        pl.kernel,
        out_type=jax.ShapeDtypeStruct((_KBN, _C2P), jnp.float32),
        mesh=mesh,
    )
    def gk(v_hbm, i_hbm, o_hbm):
        def body(i_vmem, o_vmem):
            pltpu.sync_copy(v_hbm.at[i_vmem.at[0]], o_vmem)

        pltpu.emit_pipeline(
            body,
            grid=(_KBN // _GW,),
            in_specs=[pl.BlockSpec((1, _GW), index_map=lambda i: (0, i))],
            out_specs=[pl.BlockSpec((_GW, _C2P), index_map=lambda i: (i, 0))],
            core_axis_name=("core", "subcore"),
            dimension_semantics=(pltpu.PARALLEL,),
        )(i_hbm, o_hbm)

    return gk(v_flat, idx2)


def _head_body(g_ref, x_ref, au_ref, bg_ref, w1_ref, b1_ref, g1_ref, be1_ref,
               w2_ref, b2_ref, g2_ref, be2_ref, o_ref):
    g = g_ref[...]
    vm = jnp.maximum(
        jnp.maximum(g[:, 0 * _C2P:1 * _C2P], g[:, 1 * _C2P:2 * _C2P]),
        jnp.maximum(g[:, 2 * _C2P:3 * _C2P], g[:, 3 * _C2P:4 * _C2P]),
    )
    u = jnp.dot(x_ref[...], au_ref[...],
                preferred_element_type=jnp.float32) + bg_ref[...]
    m = jnp.maximum(u + vm, 0.0)
    ones = jnp.ones((1, _BN), jnp.float32)
    z1 = jnp.dot(m, w1_ref[...], preferred_element_type=jnp.float32) + b1_ref[...]
    mu1 = jnp.dot(ones, z1, preferred_element_type=jnp.float32) * (1.0 / _BN)
    d1 = z1 - mu1
    va1 = jnp.dot(ones, d1 * d1, preferred_element_type=jnp.float32) * (1.0 / _BN)
    y1 = jnp.maximum(g1_ref[...] * d1 / jnp.sqrt(va1 + 1e-5) + be1_ref[...], 0.0)
    z2 = jnp.dot(y1, w2_ref[...], preferred_element_type=jnp.float32) + b2_ref[...]
    mu2 = jnp.dot(ones, z2, preferred_element_type=jnp.float32) * (1.0 / _BN)
    d2 = z2 - mu2
    va2 = jnp.dot(ones, d2 * d2, preferred_element_type=jnp.float32) * (1.0 / _BN)
    y2 = jnp.maximum(g2_ref[...] * d2 / jnp.sqrt(va2 + 1e-5) + be2_ref[...], 0.0)
    o_ref[...] = y2 + y1


def _head(g, x2, au, bg2, w1t, b1r, g1r, be1r, w2t, b2r, g2r, be2r):
    return pl.pallas_call(
        _head_body,
        out_shape=jax.ShapeDtypeStruct((_BN, _C), jnp.float32),
    )(g, x2, au, bg2, w1t, b1r, g1r, be1r, w2t, b2r, g2r, be2r)


def kernel(inputs, pos_embed, Wg, bg, W1, b1, g1, be1, W2, b2, g2, be2):
    xin = inputs.reshape(_B, _C, _N).transpose(0, 2, 1)
    pad = _C2P - _C2
    au = jnp.pad((Wg[:, :_C] - Wg[:, _C:]).T, ((0, 0), (0, pad)))
    av = jnp.pad(Wg[:, _C:].T, ((0, 0), (0, pad)))
    x, xn, sq, v = _prep(xin, pos_embed, av)
    sqt = jnp.swapaxes(sq, 1, 2)
    idx = _knn(xn, sqt)
    gathered = _sc_gather(v.reshape(_BN, _C2P), idx.reshape(_KBN))
    out = _head(
        gathered.reshape(_BN, _K * _C2P), x.reshape(_BN, _C), au,
        jnp.pad(bg, (0, pad)).reshape(1, _C2P),
        jnp.pad(W1.T, ((0, pad), (0, 0))), b1.reshape(1, _C), g1.reshape(1, _C), be1.reshape(1, _C),
        W2.T, b2.reshape(1, _C), g2.reshape(1, _C), be2.reshape(1, _C),
    )
    return out.reshape(_B, _N, _C).transpose(0, 2, 1).reshape(_B, _C, _T, _H, _W)


# speedup vs baseline: 1.2055x; 1.2055x over previous
import functools

import jax
import jax.numpy as jnp
from jax import lax
from jax.experimental import pallas as pl
from jax.experimental.pallas import tpu as pltpu
from jax.experimental.pallas import tpu_sc as plsc

_B, _C, _T, _H, _W = 4, 96, 8, 14, 14
_K = 4
_N = _T * _H * _W
_C2 = 2 * _C
_C2P = 256
_BN = _B * _N
_KBN = _K * _BN
_RB = 224
_NRB = _N // _RB
_GW = 128


def _prep_body(xin_ref, pe_ref, av_ref, x_ref, xn_ref, sq_ref, v_ref):
    x = xin_ref[0] + pe_ref[0]
    x_ref[0] = x
    nrm = jnp.sqrt(jnp.sum(x * x, axis=1, keepdims=True))
    xn = x / jnp.maximum(nrm, 1e-12)
    xn_ref[0] = xn
    sq_ref[0] = jnp.sum(xn * xn, axis=1, keepdims=True)
    v_ref[0] = jnp.dot(x, av_ref[...], preferred_element_type=jnp.float32)


def _prep(xin, pe, av):
    return pl.pallas_call(
        _prep_body,
        grid=(_B,),
        in_specs=[
            pl.BlockSpec((1, _N, _C), lambda b: (b, 0, 0)),
            pl.BlockSpec((1, _N, _C), lambda b: (0, 0, 0)),
            pl.BlockSpec((_C, _C2P), lambda b: (0, 0)),
        ],
        out_specs=[
            pl.BlockSpec((1, _N, _C), lambda b: (b, 0, 0)),
            pl.BlockSpec((1, _N, _C), lambda b: (b, 0, 0)),
            pl.BlockSpec((1, _N, 1), lambda b: (b, 0, 0)),
            pl.BlockSpec((1, _N, _C2P), lambda b: (b, 0, 0)),
        ],
        out_shape=[
            jax.ShapeDtypeStruct((_B, _N, _C), jnp.float32),
            jax.ShapeDtypeStruct((_B, _N, _C), jnp.float32),
            jax.ShapeDtypeStruct((_B, _N, 1), jnp.float32),
            jax.ShapeDtypeStruct((_B, _N, _C2P), jnp.float32),
        ],
    )(xin, pe, av)


def _knn_body(xb_ref, xn_ref, sqt_ref, idx_ref):
    b = pl.program_id(0)
    xb = xb_ref[0]
    xn = xn_ref[0]
    dot = lax.dot_general(xb, xn, (((1,), (1,)), ((), ())),
                          preferred_element_type=jnp.float32)
    d = sqt_ref[0] - 2.0 * dot
    iot = lax.broadcasted_iota(jnp.int32, (_RB, _N), 1)
    cols = []
    for _ in range(_K):
        mv = jnp.min(d, axis=1, keepdims=True)
        ik = jnp.min(jnp.where(d == mv, iot, _N), axis=1, keepdims=True)
        cols.append(ik)
        d = jnp.where(iot == ik, jnp.inf, d)
    idx_ref[0] = jnp.concatenate(cols, axis=1) + b * _N


def _knn(xn, sqt):
    return pl.pallas_call(
        _knn_body,
        grid=(_B, _NRB),
        in_specs=[
            pl.BlockSpec((1, _RB, _C), lambda b, i: (b, i, 0)),
            pl.BlockSpec((1, _N, _C), lambda b, i: (b, 0, 0)),
            pl.BlockSpec((1, 1, _N), lambda b, i: (b, 0, 0)),
        ],
        out_specs=pl.BlockSpec((1, _RB, _K), lambda b, i: (b, i, 0)),
        out_shape=jax.ShapeDtypeStruct((_B, _N, _K), jnp.int32),
    )(xn, xn, sqt)


def _sc_gather(v_flat, idx_flat):
    idx2 = idx_flat.reshape(1, _KBN)
    mesh = plsc.VectorSubcoreMesh(core_axis_name="core", subcore_axis_name="subcore")

    @functools.partial(
        pl.kernel,
        out_type=jax.ShapeDtypeStruct((_KBN, _C2P), jnp.float32),
        mesh=mesh,
    )
    def gk(v_hbm, i_hbm, o_hbm):
        def body(i_vmem, o_vmem):
            pltpu.sync_copy(v_hbm.at[i_vmem.at[0]], o_vmem)

        pltpu.emit_pipeline(
            body,
            grid=(_KBN // _GW,),
            in_specs=[pl.BlockSpec((1, _GW), index_map=lambda i: (0, i))],
            out_specs=[pl.BlockSpec((_GW, _C2P), index_map=lambda i: (i, 0))],
            core_axis_name=("core", "subcore"),
            dimension_semantics=(pltpu.PARALLEL,),
        )(i_hbm, o_hbm)

    return gk(v_flat, idx2)


def _head_body(g_ref, x_ref, au_ref, bg_ref, w1_ref, b1_ref, g1_ref, be1_ref,
               w2_ref, b2_ref, g2_ref, be2_ref, o_ref):
    vm = jnp.max(g_ref[...], axis=0)
    u = jnp.dot(x_ref[...], au_ref[...],
                preferred_element_type=jnp.float32) + bg_ref[...]
    m = jnp.maximum(u + vm, 0.0)
    ones = jnp.ones((1, _BN), jnp.float32)
    z1 = jnp.dot(m, w1_ref[...], preferred_element_type=jnp.float32) + b1_ref[...]
    mu1 = jnp.dot(ones, z1, preferred_element_type=jnp.float32) * (1.0 / _BN)
    d1 = z1 - mu1
    va1 = jnp.dot(ones, d1 * d1, preferred_element_type=jnp.float32) * (1.0 / _BN)
    y1 = jnp.maximum(g1_ref[...] * d1 / jnp.sqrt(va1 + 1e-5) + be1_ref[...], 0.0)
    z2 = jnp.dot(y1, w2_ref[...], preferred_element_type=jnp.float32) + b2_ref[...]
    mu2 = jnp.dot(ones, z2, preferred_element_type=jnp.float32) * (1.0 / _BN)
    d2 = z2 - mu2
    va2 = jnp.dot(ones, d2 * d2, preferred_element_type=jnp.float32) * (1.0 / _BN)
    y2 = jnp.maximum(g2_ref[...] * d2 / jnp.sqrt(va2 + 1e-5) + be2_ref[...], 0.0)
    o_ref[...] = y2 + y1


def _head(g, x2, au, bg2, w1t, b1r, g1r, be1r, w2t, b2r, g2r, be2r):
    return pl.pallas_call(
        _head_body,
        out_shape=jax.ShapeDtypeStruct((_BN, _C), jnp.float32),
    )(g, x2, au, bg2, w1t, b1r, g1r, be1r, w2t, b2r, g2r, be2r)


def kernel(inputs, pos_embed, Wg, bg, W1, b1, g1, be1, W2, b2, g2, be2):
    xin = inputs.reshape(_B, _C, _N).transpose(0, 2, 1)
    pad = _C2P - _C2
    au = jnp.pad((Wg[:, :_C] - Wg[:, _C:]).T, ((0, 0), (0, pad)))
    av = jnp.pad(Wg[:, _C:].T, ((0, 0), (0, pad)))
    x, xn, sq, v = _prep(xin, pos_embed, av)
    sqt = jnp.swapaxes(sq, 1, 2)
    idx = _knn(xn, sqt)
    gathered = _sc_gather(v.reshape(_BN, _C2P), idx.transpose(2, 0, 1).reshape(_KBN))
    out = _head(
        gathered.reshape(_K, _BN, _C2P), x.reshape(_BN, _C), au,
        jnp.pad(bg, (0, pad)).reshape(1, _C2P),
        jnp.pad(W1.T, ((0, pad), (0, 0))), b1.reshape(1, _C), g1.reshape(1, _C), be1.reshape(1, _C),
        W2.T, b2.reshape(1, _C), g2.reshape(1, _C), be2.reshape(1, _C),
    )
    return out.reshape(_B, _N, _C).transpose(0, 2, 1).reshape(_B, _C, _T, _H, _W)


# speedup vs baseline: 1.2272x; 1.0180x over previous
import functools

import jax
import jax.numpy as jnp
from jax import lax
from jax.experimental import pallas as pl
from jax.experimental.pallas import tpu as pltpu
from jax.experimental.pallas import tpu_sc as plsc

_B, _C, _T, _H, _W = 4, 96, 8, 14, 14
_K = 4
_N = _T * _H * _W
_C2 = 2 * _C
_C2P = 256
_BN = _B * _N
_KBN = _K * _BN
_RB = 224
_NRB = _N // _RB
_GW = 128


def _ab_body(xin_ref, pe_ref, av_ref, x_ref, v_ref, idx_ref, xn_s, sq_s):
    b = pl.program_id(0)
    i = pl.program_id(1)

    @pl.when(i == 0)
    def _():
        x = xin_ref[0].T + pe_ref[0]
        x_ref[0] = x
        nrm = jnp.sqrt(jnp.sum(x * x, axis=1, keepdims=True))
        xn = x / jnp.maximum(nrm, 1e-12)
        xn_s[...] = xn
        ones = jnp.ones((1, _C), jnp.float32)
        sq_s[...] = lax.dot_general(ones, xn * xn, (((1,), (1,)), ((), ())),
                                    precision=lax.Precision.HIGHEST,
                                    preferred_element_type=jnp.float32)
        v_ref[0] = jnp.dot(x, av_ref[...], preferred_element_type=jnp.float32)

    @pl.when(i > 0)
    def _():
        off = pl.multiple_of((i - 1) * _RB, _RB)
        xb = xn_s[pl.ds(off, _RB), :]
        dt = lax.dot_general(xb, xn_s[...], (((1,), (1,)), ((), ())),
                             preferred_element_type=jnp.float32)
        d = sq_s[...] - 2.0 * dt
        iot = lax.broadcasted_iota(jnp.int32, (_RB, _N), 1)
        cols = []
        for _ in range(_K):
            mv = jnp.min(d, axis=1, keepdims=True)
            ik = jnp.min(jnp.where(d == mv, iot, _N), axis=1, keepdims=True)
            cols.append(ik)
            d = jnp.where(iot == ik, jnp.inf, d)
        idx_ref[0] = jnp.concatenate(cols, axis=1) + b * _N


def _graph(xin3, pe, av):
    return pl.pallas_call(
        _ab_body,
        grid=(_B, _NRB + 1),
        in_specs=[
            pl.BlockSpec((1, _C, _N), lambda b, i: (b, 0, 0)),
            pl.BlockSpec((1, _N, _C), lambda b, i: (0, 0, 0)),
            pl.BlockSpec((_C, _C2P), lambda b, i: (0, 0)),
        ],
        out_specs=[
            pl.BlockSpec((1, _N, _C), lambda b, i: (b, 0, 0)),
            pl.BlockSpec((1, _N, _C2P), lambda b, i: (b, 0, 0)),
            pl.BlockSpec((1, _RB, _K), lambda b, i: (b, jnp.maximum(i - 1, 0), 0)),
        ],
        out_shape=[
            jax.ShapeDtypeStruct((_B, _N, _C), jnp.float32),
            jax.ShapeDtypeStruct((_B, _N, _C2P), jnp.float32),
            jax.ShapeDtypeStruct((_B, _N, _K), jnp.int32),
        ],
        scratch_shapes=[
            pltpu.VMEM((_N, _C), jnp.float32),
            pltpu.VMEM((1, _N), jnp.float32),
        ],
    )(xin3, pe, av)


def _sc_gather(v_flat, idx_flat):
    idx2 = idx_flat.reshape(1, _KBN)
    mesh = plsc.VectorSubcoreMesh(core_axis_name="core", subcore_axis_name="subcore")

    @functools.partial(
        pl.kernel,
        out_type=jax.ShapeDtypeStruct((_KBN, _C2P), jnp.float32),
        mesh=mesh,
    )
    def gk(v_hbm, i_hbm, o_hbm):
        def body(i_vmem, o_vmem):
            pltpu.sync_copy(v_hbm.at[i_vmem.at[0]], o_vmem)

        pltpu.emit_pipeline(
            body,
            grid=(_KBN // _GW,),
            in_specs=[pl.BlockSpec((1, _GW), index_map=lambda i: (0, i))],
            out_specs=[pl.BlockSpec((_GW, _C2P), index_map=lambda i: (i, 0))],
            core_axis_name=("core", "subcore"),
            dimension_semantics=(pltpu.PARALLEL,),
        )(i_hbm, o_hbm)

    return gk(v_flat, idx2)


def _head_body(g_ref, x_ref, au_ref, bg_ref, w1_ref, b1_ref, g1_ref, be1_ref,
               w2_ref, b2_ref, g2_ref, be2_ref, o_ref):
    vm = jnp.max(g_ref[...], axis=0)
    u = jnp.dot(x_ref[...], au_ref[...],
                preferred_element_type=jnp.float32) + bg_ref[...]
    m = jnp.maximum(u + vm, 0.0)
    ones = jnp.ones((1, _BN), jnp.float32)
    z1 = jnp.dot(m, w1_ref[...], preferred_element_type=jnp.float32) + b1_ref[...]
    mu1 = jnp.dot(ones, z1, preferred_element_type=jnp.float32) * (1.0 / _BN)
    d1 = z1 - mu1
    va1 = jnp.dot(ones, d1 * d1, preferred_element_type=jnp.float32) * (1.0 / _BN)
    y1 = jnp.maximum(g1_ref[...] * d1 / jnp.sqrt(va1 + 1e-5) + be1_ref[...], 0.0)
    z2 = jnp.dot(y1, w2_ref[...], preferred_element_type=jnp.float32) + b2_ref[...]
    mu2 = jnp.dot(ones, z2, preferred_element_type=jnp.float32) * (1.0 / _BN)
    d2 = z2 - mu2
    va2 = jnp.dot(ones, d2 * d2, preferred_element_type=jnp.float32) * (1.0 / _BN)
    y2 = jnp.maximum(g2_ref[...] * d2 / jnp.sqrt(va2 + 1e-5) + be2_ref[...], 0.0)
    o_ref[...] = y2 + y1


def _head(g, x2, au, bg2, w1t, b1r, g1r, be1r, w2t, b2r, g2r, be2r):
    return pl.pallas_call(
        _head_body,
        out_shape=jax.ShapeDtypeStruct((_BN, _C), jnp.float32),
    )(g, x2, au, bg2, w1t, b1r, g1r, be1r, w2t, b2r, g2r, be2r)


def kernel(inputs, pos_embed, Wg, bg, W1, b1, g1, be1, W2, b2, g2, be2):
    xin3 = inputs.reshape(_B, _C, _N)
    pad = _C2P - _C2
    au = jnp.pad((Wg[:, :_C] - Wg[:, _C:]).T, ((0, 0), (0, pad)))
    av = jnp.pad(Wg[:, _C:].T, ((0, 0), (0, pad)))
    x, v, idx = _graph(xin3, pos_embed, av)
    gathered = _sc_gather(v.reshape(_BN, _C2P), idx.transpose(2, 0, 1).reshape(_KBN))
    out = _head(
        gathered.reshape(_K, _BN, _C2P), x.reshape(_BN, _C), au,
        jnp.pad(bg, (0, pad)).reshape(1, _C2P),
        jnp.pad(W1.T, ((0, pad), (0, 0))), b1.reshape(1, _C), g1.reshape(1, _C), be1.reshape(1, _C),
        W2.T, b2.reshape(1, _C), g2.reshape(1, _C), be2.reshape(1, _C),
    )
    return out.reshape(_B, _N, _C).transpose(0, 2, 1).reshape(_B, _C, _T, _H, _W)


# speedup vs baseline: 1.2369x; 1.0078x over previous
import functools

import jax
import jax.numpy as jnp
from jax import lax
from jax.experimental import pallas as pl
from jax.experimental.pallas import tpu as pltpu
from jax.experimental.pallas import tpu_sc as plsc

_B, _C, _T, _H, _W = 4, 96, 8, 14, 14
_K = 4
_N = _T * _H * _W
_C2 = 2 * _C
_C2P = 256
_BN = _B * _N
_KBN = _K * _BN
_RB = 392
_NRB = _N // _RB
_GW = 128


def _ab_body(xin_ref, pe_ref, av_ref, x_ref, v_ref, idx_ref, xn_s, sq_s):
    b = pl.program_id(0)
    i = pl.program_id(1)

    @pl.when(i == 0)
    def _():
        x = xin_ref[0].T + pe_ref[0]
        x_ref[0] = x
        nrm = jnp.sqrt(jnp.sum(x * x, axis=1, keepdims=True))
        xn = x / jnp.maximum(nrm, 1e-12)
        xn_s[...] = xn
        ones = jnp.ones((1, _C), jnp.float32)
        sq_s[...] = lax.dot_general(ones, xn * xn, (((1,), (1,)), ((), ())),
                                    precision=lax.Precision.HIGHEST,
                                    preferred_element_type=jnp.float32)
        v_ref[0] = jnp.dot(x, av_ref[...], precision=lax.Precision.HIGHEST,
                           preferred_element_type=jnp.float32)

    @pl.when(i > 0)
    def _():
        off = pl.multiple_of((i - 1) * _RB, _RB)
        xb = xn_s[pl.ds(off, _RB), :]
        dt = lax.dot_general(xb, xn_s[...], (((1,), (1,)), ((), ())),
                             preferred_element_type=jnp.float32)
        d = sq_s[...] - 2.0 * dt
        iot = lax.broadcasted_iota(jnp.int32, (_RB, _N), 1)
        cols = []
        for _ in range(_K):
            mv = jnp.min(d, axis=1, keepdims=True)
            ik = jnp.min(jnp.where(d == mv, iot, _N), axis=1, keepdims=True)
            cols.append(ik)
            d = jnp.where(iot == ik, jnp.inf, d)
        idx_ref[0] = jnp.concatenate(cols, axis=1) + b * _N


def _graph(xin3, pe, av):
    return pl.pallas_call(
        _ab_body,
        grid=(_B, _NRB + 1),
        in_specs=[
            pl.BlockSpec((1, _C, _N), lambda b, i: (b, 0, 0)),
            pl.BlockSpec((1, _N, _C), lambda b, i: (0, 0, 0)),
            pl.BlockSpec((_C, _C2P), lambda b, i: (0, 0)),
        ],
        out_specs=[
            pl.BlockSpec((1, _N, _C), lambda b, i: (b, 0, 0)),
            pl.BlockSpec((1, _N, _C2P), lambda b, i: (b, 0, 0)),
            pl.BlockSpec((1, _RB, _K), lambda b, i: (b, jnp.maximum(i - 1, 0), 0)),
        ],
        out_shape=[
            jax.ShapeDtypeStruct((_B, _N, _C), jnp.float32),
            jax.ShapeDtypeStruct((_B, _N, _C2P), jnp.float32),
            jax.ShapeDtypeStruct((_B, _N, _K), jnp.int32),
        ],
        scratch_shapes=[
            pltpu.VMEM((_N, _C), jnp.float32),
            pltpu.VMEM((1, _N), jnp.float32),
        ],
    )(xin3, pe, av)


def _sc_gather(v_flat, idx_flat):
    idx2 = idx_flat.reshape(1, _KBN)
    mesh = plsc.VectorSubcoreMesh(core_axis_name="core", subcore_axis_name="subcore")

    @functools.partial(
        pl.kernel,
        out_type=jax.ShapeDtypeStruct((_KBN, _C2P), jnp.float32),
        mesh=mesh,
    )
    def gk(v_hbm, i_hbm, o_hbm):
        def body(i_vmem, o_vmem):
            pltpu.sync_copy(v_hbm.at[i_vmem.at[0]], o_vmem)

        pltpu.emit_pipeline(
            body,
            grid=(_KBN // _GW,),
            in_specs=[pl.BlockSpec((1, _GW), index_map=lambda i: (0, i))],
            out_specs=[pl.BlockSpec((_GW, _C2P), index_map=lambda i: (i, 0))],
            core_axis_name=("core", "subcore"),
            dimension_semantics=(pltpu.PARALLEL,),
        )(i_hbm, o_hbm)

    return gk(v_flat, idx2)


def _head_body(g_ref, x_ref, au_ref, bg_ref, w1_ref, b1_ref, g1_ref, be1_ref,
               w2_ref, b2_ref, g2_ref, be2_ref, o_ref):
    vm = jnp.max(g_ref[...], axis=0)
    u = jnp.dot(x_ref[...], au_ref[...], precision=lax.Precision.HIGHEST,
                preferred_element_type=jnp.float32) + bg_ref[...]
    m = jnp.maximum(u + vm, 0.0)
    ones = jnp.ones((1, _BN), jnp.float32)
    z1 = jnp.dot(m, w1_ref[...], preferred_element_type=jnp.float32) + b1_ref[...]
    mu1 = jnp.dot(ones, z1, preferred_element_type=jnp.float32) * (1.0 / _BN)
    d1 = z1 - mu1
    va1 = jnp.dot(ones, d1 * d1, preferred_element_type=jnp.float32) * (1.0 / _BN)
    y1 = jnp.maximum(g1_ref[...] * d1 / jnp.sqrt(va1 + 1e-5) + be1_ref[...], 0.0)
    z2 = jnp.dot(y1, w2_ref[...], preferred_element_type=jnp.float32) + b2_ref[...]
    mu2 = jnp.dot(ones, z2, preferred_element_type=jnp.float32) * (1.0 / _BN)
    d2 = z2 - mu2
    va2 = jnp.dot(ones, d2 * d2, preferred_element_type=jnp.float32) * (1.0 / _BN)
    y2 = jnp.maximum(g2_ref[...] * d2 / jnp.sqrt(va2 + 1e-5) + be2_ref[...], 0.0)
    o_ref[...] = y2 + y1


def _head(g, x2, au, bg2, w1t, b1r, g1r, be1r, w2t, b2r, g2r, be2r):
    return pl.pallas_call(
        _head_body,
        out_shape=jax.ShapeDtypeStruct((_BN, _C), jnp.float32),
    )(g, x2, au, bg2, w1t, b1r, g1r, be1r, w2t, b2r, g2r, be2r)


def kernel(inputs, pos_embed, Wg, bg, W1, b1, g1, be1, W2, b2, g2, be2):
    xin3 = inputs.reshape(_B, _C, _N)
    pad = _C2P - _C2
    au = jnp.pad((Wg[:, :_C] - Wg[:, _C:]).T, ((0, 0), (0, pad)))
    av = jnp.pad(Wg[:, _C:].T, ((0, 0), (0, pad)))
    x, v, idx = _graph(xin3, pos_embed, av)
    gathered = _sc_gather(v.reshape(_BN, _C2P), idx.transpose(2, 0, 1).reshape(_KBN))
    out = _head(
        gathered.reshape(_K, _BN, _C2P), x.reshape(_BN, _C), au,
        jnp.pad(bg, (0, pad)).reshape(1, _C2P),
        jnp.pad(W1.T, ((0, pad), (0, 0))), b1.reshape(1, _C), g1.reshape(1, _C), be1.reshape(1, _C),
        W2.T, b2.reshape(1, _C), g2.reshape(1, _C), be2.reshape(1, _C),
    )
    return out.reshape(_B, _N, _C).transpose(0, 2, 1).reshape(_B, _C, _T, _H, _W)


# speedup vs baseline: 1.2600x; 1.0187x over previous
import functools

import jax
import jax.numpy as jnp
from jax import lax
from jax.experimental import pallas as pl
from jax.experimental.pallas import tpu as pltpu
from jax.experimental.pallas import tpu_sc as plsc

_B, _C, _T, _H, _W = 4, 96, 8, 14, 14
_K = 4
_N = _T * _H * _W
_C2 = 2 * _C
_C2P = 256
_BN = _B * _N
_KBN = _K * _BN
_RB = 392
_NRB = _N // _RB
_GW = 128


def _ab_body(xin_ref, pe_ref, av_ref, x_ref, v_ref, idx_ref, xn_s, sq_s):
    b = pl.program_id(0)
    i = pl.program_id(1)

    @pl.when(i == 0)
    def _():
        x = xin_ref[0].T + pe_ref[0]
        x_ref[0] = x
        nrm = jnp.sqrt(jnp.sum(x * x, axis=1, keepdims=True))
        xn = x / jnp.maximum(nrm, 1e-12)
        xn_s[...] = xn
        ones = jnp.ones((1, _C), jnp.float32)
        sq_s[...] = lax.dot_general(ones, xn * xn, (((1,), (1,)), ((), ())),
                                    precision=lax.Precision.HIGHEST,
                                    preferred_element_type=jnp.float32)
        v_ref[0] = jnp.dot(x, av_ref[...], precision=lax.Precision.HIGHEST,
                           preferred_element_type=jnp.float32)

    @pl.when(i > 0)
    def _():
        off = pl.multiple_of((i - 1) * _RB, _RB)
        xb = xn_s[pl.ds(off, _RB), :]
        dt = lax.dot_general(xb, xn_s[...], (((1,), (1,)), ((), ())),
                             preferred_element_type=jnp.float32)
        d = sq_s[...] - 2.0 * dt
        iot = lax.broadcasted_iota(jnp.int32, (_RB, _N), 1)
        cols = []
        for _ in range(_K):
            mv = jnp.min(d, axis=1, keepdims=True)
            ik = jnp.min(jnp.where(d == mv, iot, _N), axis=1, keepdims=True)
            cols.append(ik)
            d = jnp.where(iot == ik, jnp.inf, d)
        idx_ref[0] = jnp.concatenate(cols, axis=1) + b * _N


def _graph(xin3, pe, av):
    return pl.pallas_call(
        _ab_body,
        grid=(_B, _NRB + 1),
        in_specs=[
            pl.BlockSpec((1, _C, _N), lambda b, i: (b, 0, 0)),
            pl.BlockSpec((1, _N, _C), lambda b, i: (0, 0, 0)),
            pl.BlockSpec((_C, _C2P), lambda b, i: (0, 0)),
        ],
        out_specs=[
            pl.BlockSpec((1, _N, _C), lambda b, i: (b, 0, 0)),
            pl.BlockSpec((1, _N, _C2P), lambda b, i: (b, 0, 0)),
            pl.BlockSpec((1, _RB, _K), lambda b, i: (b, jnp.maximum(i - 1, 0), 0)),
        ],
        out_shape=[
            jax.ShapeDtypeStruct((_B, _N, _C), jnp.float32),
            jax.ShapeDtypeStruct((_B, _N, _C2P), jnp.float32),
            jax.ShapeDtypeStruct((_B, _N, _K), jnp.int32),
        ],
        scratch_shapes=[
            pltpu.VMEM((_N, _C), jnp.float32),
            pltpu.VMEM((1, _N), jnp.float32),
        ],
    )(xin3, pe, av)


def _sc_gather(v_flat, idx_flat):
    idx2 = idx_flat.reshape(1, _KBN)
    mesh = plsc.VectorSubcoreMesh(core_axis_name="core", subcore_axis_name="subcore")

    @functools.partial(
        pl.kernel,
        out_type=jax.ShapeDtypeStruct((_KBN, _C2P), jnp.float32),
        mesh=mesh,
    )
    def gk(v_hbm, i_hbm, o_hbm):
        def body(i_vmem, o_vmem):
            pltpu.sync_copy(v_hbm.at[i_vmem.at[0]], o_vmem)

        pltpu.emit_pipeline(
            body,
            grid=(_KBN // _GW,),
            in_specs=[pl.BlockSpec((1, _GW), index_map=lambda i: (0, i))],
            out_specs=[pl.BlockSpec((_GW, _C2P), index_map=lambda i: (i, 0))],
            core_axis_name=("core", "subcore"),
            dimension_semantics=(pltpu.PARALLEL,),
        )(i_hbm, o_hbm)

    return gk(v_flat, idx2)


def _head_body(g_ref, x_ref, au_ref, bg_ref, w1_ref, b1_ref, g1_ref, be1_ref,
               w2_ref, b2_ref, g2_ref, be2_ref, o_ref):
    vm = jnp.max(g_ref[...], axis=0)
    u = jnp.dot(x_ref[...], au_ref[...], precision=lax.Precision.HIGHEST,
                preferred_element_type=jnp.float32) + bg_ref[...]
    m = jnp.maximum(u + vm, 0.0)
    ones = jnp.ones((1, _BN), jnp.float32)
    z1 = jnp.dot(m, w1_ref[...], preferred_element_type=jnp.float32) + b1_ref[...]
    mu1 = jnp.dot(ones, z1, preferred_element_type=jnp.float32) * (1.0 / _BN)
    d1 = z1 - mu1
    va1 = jnp.dot(ones, d1 * d1, preferred_element_type=jnp.float32) * (1.0 / _BN)
    y1 = jnp.maximum(g1_ref[...] * d1 / jnp.sqrt(va1 + 1e-5) + be1_ref[...], 0.0)
    z2 = jnp.dot(y1, w2_ref[...], preferred_element_type=jnp.float32) + b2_ref[...]
    mu2 = jnp.dot(ones, z2, preferred_element_type=jnp.float32) * (1.0 / _BN)
    d2 = z2 - mu2
    va2 = jnp.dot(ones, d2 * d2, preferred_element_type=jnp.float32) * (1.0 / _BN)
    y2 = jnp.maximum(g2_ref[...] * d2 / jnp.sqrt(va2 + 1e-5) + be2_ref[...], 0.0)
    y = y2 + y1
    for bb in range(_B):
        o_ref[bb] = y[bb * _N:(bb + 1) * _N, :].T


def _head(g, x2, au, bg2, w1t, b1r, g1r, be1r, w2t, b2r, g2r, be2r):
    return pl.pallas_call(
        _head_body,
        out_shape=jax.ShapeDtypeStruct((_B, _C, _N), jnp.float32),
    )(g, x2, au, bg2, w1t, b1r, g1r, be1r, w2t, b2r, g2r, be2r)


def kernel(inputs, pos_embed, Wg, bg, W1, b1, g1, be1, W2, b2, g2, be2):
    xin3 = inputs.reshape(_B, _C, _N)
    pad = _C2P - _C2
    au = jnp.pad((Wg[:, :_C] - Wg[:, _C:]).T, ((0, 0), (0, pad)))
    av = jnp.pad(Wg[:, _C:].T, ((0, 0), (0, pad)))
    x, v, idx = _graph(xin3, pos_embed, av)
    gathered = _sc_gather(v.reshape(_BN, _C2P), idx.transpose(2, 0, 1).reshape(_KBN))
    out = _head(
        gathered.reshape(_K, _BN, _C2P), x.reshape(_BN, _C), au,
        jnp.pad(bg, (0, pad)).reshape(1, _C2P),
        jnp.pad(W1.T, ((0, pad), (0, 0))), b1.reshape(1, _C), g1.reshape(1, _C), be1.reshape(1, _C),
        W2.T, b2.reshape(1, _C), g2.reshape(1, _C), be2.reshape(1, _C),
    )
    return out.reshape(_B, _C, _T, _H, _W)


# speedup vs baseline: 1.2786x; 1.0148x over previous
import functools

import jax
import jax.numpy as jnp
from jax import lax
from jax.experimental import pallas as pl
from jax.experimental.pallas import tpu as pltpu
from jax.experimental.pallas import tpu_sc as plsc

_B, _C, _T, _H, _W = 4, 96, 8, 14, 14
_K = 4
_N = _T * _H * _W
_C2 = 2 * _C
_C2P = 256
_BN = _B * _N
_KBN = _K * _BN
_RB = 784
_NRB = _N // _RB
_GW = 128


def _ab_body(xin_ref, pe_ref, av_ref, x_ref, v_ref, idx_ref, xn_s, sq_s):
    b = pl.program_id(0)
    i = pl.program_id(1)

    @pl.when(i == 0)
    def _():
        x = xin_ref[0].T + pe_ref[0]
        x_ref[0] = x
        nrm = jnp.sqrt(jnp.sum(x * x, axis=1, keepdims=True))
        xn = x / jnp.maximum(nrm, 1e-12)
        xn_s[...] = xn
        ones = jnp.ones((1, _C), jnp.float32)
        sq_s[...] = lax.dot_general(ones, xn * xn, (((1,), (1,)), ((), ())),
                                    precision=lax.Precision.HIGHEST,
                                    preferred_element_type=jnp.float32)
        v_ref[0] = jnp.dot(x, av_ref[...], precision=lax.Precision.HIGHEST,
                           preferred_element_type=jnp.float32)

    @pl.when(i > 0)
    def _():
        off = pl.multiple_of((i - 1) * _RB, _RB)
        xb = xn_s[pl.ds(off, _RB), :]
        dt = lax.dot_general(xb, xn_s[...], (((1,), (1,)), ((), ())),
                             preferred_element_type=jnp.float32)
        d = sq_s[...] - 2.0 * dt
        iot = lax.broadcasted_iota(jnp.int32, (_RB, _N), 1)
        cols = []
        for _ in range(_K):
            mv = jnp.min(d, axis=1, keepdims=True)
            ik = jnp.min(jnp.where(d == mv, iot, _N), axis=1, keepdims=True)
            cols.append(ik)
            d = jnp.where(iot == ik, jnp.inf, d)
        idx_ref[0] = jnp.concatenate(cols, axis=1) + b * _N


def _graph(xin3, pe, av):
    return pl.pallas_call(
        _ab_body,
        grid=(_B, _NRB + 1),
        in_specs=[
            pl.BlockSpec((1, _C, _N), lambda b, i: (b, 0, 0)),
            pl.BlockSpec((1, _N, _C), lambda b, i: (0, 0, 0)),
            pl.BlockSpec((_C, _C2P), lambda b, i: (0, 0)),
        ],
        out_specs=[
            pl.BlockSpec((1, _N, _C), lambda b, i: (b, 0, 0)),
            pl.BlockSpec((1, _N, _C2P), lambda b, i: (b, 0, 0)),
            pl.BlockSpec((1, _RB, _K), lambda b, i: (b, jnp.maximum(i - 1, 0), 0)),
        ],
        out_shape=[
            jax.ShapeDtypeStruct((_B, _N, _C), jnp.float32),
            jax.ShapeDtypeStruct((_B, _N, _C2P), jnp.float32),
            jax.ShapeDtypeStruct((_B, _N, _K), jnp.int32),
        ],
        scratch_shapes=[
            pltpu.VMEM((_N, _C), jnp.float32),
            pltpu.VMEM((1, _N), jnp.float32),
        ],
    )(xin3, pe, av)


def _sc_gather(v_flat, idx_flat):
    idx2 = idx_flat.reshape(1, _KBN)
    mesh = plsc.VectorSubcoreMesh(core_axis_name="core", subcore_axis_name="subcore")

    @functools.partial(
        pl.kernel,
        out_type=jax.ShapeDtypeStruct((_KBN, _C2P), jnp.float32),
        mesh=mesh,
    )
    def gk(v_hbm, i_hbm, o_hbm):
        def body(i_vmem, o_vmem):
            pltpu.sync_copy(v_hbm.at[i_vmem.at[0]], o_vmem)

        pltpu.emit_pipeline(
            body,
            grid=(_KBN // _GW,),
            in_specs=[pl.BlockSpec((1, _GW), index_map=lambda i: (0, i))],
            out_specs=[pl.BlockSpec((_GW, _C2P), index_map=lambda i: (i, 0))],
            core_axis_name=("core", "subcore"),
            dimension_semantics=(pltpu.PARALLEL,),
        )(i_hbm, o_hbm)

    return gk(v_flat, idx2)


def _head_body(g_ref, x_ref, au_ref, bg_ref, w1_ref, b1_ref, g1_ref, be1_ref,
               w2_ref, b2_ref, g2_ref, be2_ref, o_ref):
    vm = jnp.max(g_ref[...], axis=0)
    u = jnp.dot(x_ref[...], au_ref[...], precision=lax.Precision.HIGHEST,
                preferred_element_type=jnp.float32) + bg_ref[...]
    m = jnp.maximum(u + vm, 0.0)
    ones = jnp.ones((1, _BN), jnp.float32)
    z1 = jnp.dot(m, w1_ref[...], preferred_element_type=jnp.float32) + b1_ref[...]
    mu1 = jnp.dot(ones, z1, preferred_element_type=jnp.float32) * (1.0 / _BN)
    d1 = z1 - mu1
    va1 = jnp.dot(ones, d1 * d1, preferred_element_type=jnp.float32) * (1.0 / _BN)
    y1 = jnp.maximum(g1_ref[...] * d1 / jnp.sqrt(va1 + 1e-5) + be1_ref[...], 0.0)
    z2 = jnp.dot(y1, w2_ref[...], preferred_element_type=jnp.float32) + b2_ref[...]
    mu2 = jnp.dot(ones, z2, preferred_element_type=jnp.float32) * (1.0 / _BN)
    d2 = z2 - mu2
    va2 = jnp.dot(ones, d2 * d2, preferred_element_type=jnp.float32) * (1.0 / _BN)
    y2 = jnp.maximum(g2_ref[...] * d2 / jnp.sqrt(va2 + 1e-5) + be2_ref[...], 0.0)
    y = y2 + y1
    for bb in range(_B):
        o_ref[bb] = y[bb * _N:(bb + 1) * _N, :].T


def _head(g, x2, au, bg2, w1t, b1r, g1r, be1r, w2t, b2r, g2r, be2r):
    return pl.pallas_call(
        _head_body,
        out_shape=jax.ShapeDtypeStruct((_B, _C, _N), jnp.float32),
    )(g, x2, au, bg2, w1t, b1r, g1r, be1r, w2t, b2r, g2r, be2r)


def kernel(inputs, pos_embed, Wg, bg, W1, b1, g1, be1, W2, b2, g2, be2):
    xin3 = inputs.reshape(_B, _C, _N)
    pad = _C2P - _C2
    au = jnp.pad((Wg[:, :_C] - Wg[:, _C:]).T, ((0, 0), (0, pad)))
    av = jnp.pad(Wg[:, _C:].T, ((0, 0), (0, pad)))
    x, v, idx = _graph(xin3, pos_embed, av)
    gathered = _sc_gather(v.reshape(_BN, _C2P), idx.transpose(2, 0, 1).reshape(_KBN))
    out = _head(
        gathered.reshape(_K, _BN, _C2P), x.reshape(_BN, _C), au,
        jnp.pad(bg, (0, pad)).reshape(1, _C2P),
        jnp.pad(W1.T, ((0, pad), (0, 0))), b1.reshape(1, _C), g1.reshape(1, _C), be1.reshape(1, _C),
        W2.T, b2.reshape(1, _C), g2.reshape(1, _C), be2.reshape(1, _C),
    )
    return out.reshape(_B, _C, _T, _H, _W)
